# baseline (device time: 336438 ns/iter reference)
import jax
import jax.numpy as jnp
from jax import lax
from jax.experimental import pallas as pl
from jax.experimental.pallas import tpu as pltpu

M = 8192
D = 2048
BLOCK_M = 512
N_BLOCKS = M // BLOCK_M
QUARTER = BLOCK_M // 4
HALF_Q = QUARTER // 2
N_SLOTS = 5
EPS = 1e-6

SEM_Z = 0
SEM_FULL_L = 1
SEM_FULL_R = 2
SEM_HALF_L = 3
SEM_HALF_R = 4
SEM_Z2 = 5
N_FLOWS = 6
Z_EXTRA = 40
RING_A = 48


def kernel(partial, resid, gamma):
    partial = partial.reshape(M, D)
    gamma = gamma.reshape(1, D)

    def body(partial_ref, resid_ref, gamma_ref, partial_cmp_ref, out_ref,
             asm, send_sems, recv_sems):
        c = pl.program_id(0)
        my_x = lax.axis_index("x")
        my_y = lax.axis_index("y")
        my_z = lax.axis_index("z")

        p = my_x + 3 * my_y - 2 * my_x * my_y

        def ring_coords(q):
            qh = q // 2
            ql = lax.rem(q, 4) % 2
            return (qh + ql - 2 * qh * ql, qh)

        pr = lax.rem(p + 1, 4)
        plft = lax.rem(p + 3, 4)
        rx, ry = ring_coords(pr)
        lx, ly = ring_coords(plft)
        right_dev = (rx, ry, my_z)
        left_dev = (lx, ly, my_z)
        zpeer_dev = (my_x, my_y, 1 - my_z)

        s0 = lax.rem(c, N_SLOTS)
        s1 = lax.rem(c + N_SLOTS - 1, N_SLOTS)
        s2 = lax.rem(c + N_SLOTS - 2, N_SLOTS)
        s3 = lax.rem(c + N_SLOTS - 3, N_SLOTS)

        def quarter_rows(ref_slot, q, off, rows):
            return ref_slot.at[pl.ds(q * QUARTER + off, rows), :]

        def copy(src, dst, ssem, dev, rsem_idx, slot):
            return pltpu.make_async_remote_copy(
                src_ref=src,
                dst_ref=dst,
                send_sem=send_sems.at[ssem],
                recv_sem=recv_sems.at[slot, rsem_idx],
                device_id=dev,
                device_id_type=pl.DeviceIdType.MESH,
            )

        popp = lax.rem(p + 2, 4)

        @pl.when(c < N_BLOCKS)
        def _():
            r = copy(quarter_rows(partial_ref, p, 0, QUARTER),
                     quarter_rows(asm.at[s0], p, 0, QUARTER),
                     0, zpeer_dev, SEM_Z, s0)
            r.start()
            r2 = copy(quarter_rows(partial_ref, popp, 0, Z_EXTRA),
                      quarter_rows(asm.at[s0], popp, 0, Z_EXTRA),
                      5, zpeer_dev, SEM_Z2, s0)
            r2.start()

        @pl.when(jnp.logical_and(c >= 1, c <= N_BLOCKS))
        def _():
            copy(quarter_rows(asm.at[s1], p, 0, QUARTER),
                 quarter_rows(asm.at[s1], p, 0, QUARTER),
                 0, zpeer_dev, SEM_Z, s1).wait_recv()

        @pl.when(jnp.logical_and(c >= 2, c <= N_BLOCKS + 1))
        def _():
            copy(quarter_rows(asm.at[s2], plft, 0, QUARTER),
                 quarter_rows(asm.at[s2], plft, 0, QUARTER),
                 0, left_dev, SEM_FULL_L, s2).wait_recv()
            copy(quarter_rows(asm.at[s2], pr, 0, QUARTER),
                 quarter_rows(asm.at[s2], pr, 0, QUARTER),
                 0, right_dev, SEM_FULL_R, s2).wait_recv()


        @pl.when(jnp.logical_and(c >= 1, c <= N_BLOCKS))
        def _():
            r1 = copy(quarter_rows(asm.at[s1], p, 0, QUARTER),
                      quarter_rows(asm.at[s1], p, 0, QUARTER),
                      1, right_dev, SEM_FULL_L, s1)
            r1.start()
            r2 = copy(quarter_rows(asm.at[s1], p, 0, QUARTER),
                      quarter_rows(asm.at[s1], p, 0, QUARTER),
                      2, left_dev, SEM_FULL_R, s1)
            r2.start()

        @pl.when(jnp.logical_and(c >= 2, c <= N_BLOCKS + 1))
        def _():
            r3 = copy(quarter_rows(asm.at[s2], plft, Z_EXTRA, RING_A),
                      quarter_rows(asm.at[s2], plft, Z_EXTRA, RING_A),
                      3, right_dev, SEM_HALF_L, s2)
            r3.start()
            r4 = copy(quarter_rows(asm.at[s2], pr, Z_EXTRA + RING_A,
                                   QUARTER - Z_EXTRA - RING_A),
                      quarter_rows(asm.at[s2], pr, Z_EXTRA + RING_A,
                                   QUARTER - Z_EXTRA - RING_A),
                      4, left_dev, SEM_HALF_R, s2)
            r4.start()

        @pl.when(jnp.logical_and(c >= 3, c <= N_BLOCKS + 2))
        def _():
            copy(quarter_rows(asm.at[s3], popp, 0, Z_EXTRA),
                 quarter_rows(asm.at[s3], popp, 0, Z_EXTRA),
                 0, zpeer_dev, SEM_Z2, s3).wait_recv()
            copy(quarter_rows(asm.at[s3], popp, Z_EXTRA, RING_A),
                 quarter_rows(asm.at[s3], popp, Z_EXTRA, RING_A),
                 0, left_dev, SEM_HALF_L, s3).wait_recv()
            copy(quarter_rows(asm.at[s3], popp, Z_EXTRA + RING_A,
                              QUARTER - Z_EXTRA - RING_A),
                 quarter_rows(asm.at[s3], popp, Z_EXTRA + RING_A,
                              QUARTER - Z_EXTRA - RING_A),
                 0, right_dev, SEM_HALF_R, s3).wait_recv()
            y = partial_cmp_ref[...] + resid_ref[...] + asm[s3]
            inv = lax.rsqrt(jnp.mean(y * y, axis=-1, keepdims=True) + EPS)
            out_ref[...] = y * inv * gamma_ref[...]

        @pl.when(c < N_BLOCKS)
        def _():
            copy(quarter_rows(partial_ref, p, 0, QUARTER),
                 quarter_rows(asm.at[s0], p, 0, QUARTER),
                 0, zpeer_dev, SEM_Z, s0).wait_send()
            copy(quarter_rows(partial_ref, popp, 0, Z_EXTRA),
                 quarter_rows(asm.at[s0], popp, 0, Z_EXTRA),
                 5, zpeer_dev, SEM_Z2, s0).wait_send()

        @pl.when(jnp.logical_and(c >= 1, c <= N_BLOCKS))
        def _():
            copy(quarter_rows(asm.at[s1], p, 0, QUARTER),
                 quarter_rows(asm.at[s1], p, 0, QUARTER),
                 1, right_dev, SEM_FULL_L, s1).wait_send()
            copy(quarter_rows(asm.at[s1], p, 0, QUARTER),
                 quarter_rows(asm.at[s1], p, 0, QUARTER),
                 2, left_dev, SEM_FULL_R, s1).wait_send()

        @pl.when(jnp.logical_and(c >= 2, c <= N_BLOCKS + 1))
        def _():
            copy(quarter_rows(asm.at[s2], plft, Z_EXTRA, RING_A),
                 quarter_rows(asm.at[s2], plft, Z_EXTRA, RING_A),
                 3, right_dev, SEM_HALF_L, s2).wait_send()
            copy(quarter_rows(asm.at[s2], pr, Z_EXTRA + RING_A,
                              QUARTER - Z_EXTRA - RING_A),
                 quarter_rows(asm.at[s2], pr, Z_EXTRA + RING_A,
                              QUARTER - Z_EXTRA - RING_A),
                 4, left_dev, SEM_HALF_R, s2).wait_send()

    def in_idx(c):
        return (jnp.minimum(c, N_BLOCKS - 1), 0)

    def out_idx(c):
        return (jnp.clip(c - 3, 0, N_BLOCKS - 1), 0)

    return pl.pallas_call(
        body,
        grid=(N_BLOCKS + 3,),
        in_specs=[
            pl.BlockSpec((BLOCK_M, D), in_idx),
            pl.BlockSpec((BLOCK_M, D), out_idx),
            pl.BlockSpec((1, D), lambda c: (0, 0)),
            pl.BlockSpec((BLOCK_M, D), out_idx),
        ],
        out_specs=pl.BlockSpec((BLOCK_M, D), out_idx),
        out_shape=jax.ShapeDtypeStruct((M, D), jnp.float32),
        scratch_shapes=[
            pltpu.VMEM((N_SLOTS, BLOCK_M, D), jnp.float32),
            pltpu.SemaphoreType.DMA((N_FLOWS,)),
            pltpu.SemaphoreType.DMA((N_SLOTS, N_FLOWS)),
        ],
        compiler_params=pltpu.CompilerParams(
            vmem_limit_bytes=100 * 1024 * 1024,
        ),
    )(partial, resid, gamma, partial)


# device time: 335281 ns/iter; 1.0035x vs baseline; 1.0035x over previous
import jax
import jax.numpy as jnp
from jax import lax
from jax.experimental import pallas as pl
from jax.experimental.pallas import tpu as pltpu

M = 8192
D = 2048
BLOCK_M = 512
N_BLOCKS = M // BLOCK_M
QUARTER = BLOCK_M // 4
SUB = QUARTER // 2
Z_EXTRA = 40
RING_A = 48
RING_B = QUARTER - Z_EXTRA - RING_A
N_SLOTS = 5
EPS = 1e-6

R_Z0 = 0
R_Z1 = 1
R_ZX = 2
R_FL0 = 3
R_FL1 = 4
R_FR0 = 5
R_FR1 = 6
R_HL = 7
R_HR = 8
N_FLOWS = 9


def kernel(partial, resid, gamma):
    partial = partial.reshape(M, D)
    gamma = gamma.reshape(1, D)

    def body(partial_ref, resid_ref, gamma_ref, partial_cmp_ref, out_ref,
             asm, send_sems, recv_sems):
        c = pl.program_id(0)
        my_x = lax.axis_index("x")
        my_y = lax.axis_index("y")
        my_z = lax.axis_index("z")

        p = my_x + 3 * my_y - 2 * my_x * my_y

        def ring_coords(q):
            qh = q // 2
            ql = lax.rem(q, 2)
            return (qh + ql - 2 * qh * ql, qh)

        pr = lax.rem(p + 1, 4)
        plft = lax.rem(p + 3, 4)
        popp = lax.rem(p + 2, 4)
        rx, ry = ring_coords(pr)
        lx, ly = ring_coords(plft)
        right_dev = (rx, ry, my_z)
        left_dev = (lx, ly, my_z)
        zpeer_dev = (my_x, my_y, 1 - my_z)

        s0 = lax.rem(c, N_SLOTS)
        s1 = lax.rem(c + N_SLOTS - 1, N_SLOTS)
        s2 = lax.rem(c + N_SLOTS - 2, N_SLOTS)
        s3 = lax.rem(c + N_SLOTS - 3, N_SLOTS)

        def rows(ref_slot, q, off, n):
            return ref_slot.at[pl.ds(q * QUARTER + off, n), :]

        def copy(src, dst, ssem, dev, rsem, slot):
            return pltpu.make_async_remote_copy(
                src_ref=src, dst_ref=dst,
                send_sem=send_sems.at[ssem],
                recv_sem=recv_sems.at[slot, rsem],
                device_id=dev, device_id_type=pl.DeviceIdType.MESH,
            )

        def z_copies():
            return [
                copy(rows(partial_ref, p, 0, SUB),
                     rows(asm.at[s0], p, 0, SUB), 0, zpeer_dev, R_Z0, s0),
                copy(rows(partial_ref, p, SUB, SUB),
                     rows(asm.at[s0], p, SUB, SUB), 1, zpeer_dev, R_Z1, s0),
                copy(rows(partial_ref, popp, 0, Z_EXTRA),
                     rows(asm.at[s0], popp, 0, Z_EXTRA),
                     2, zpeer_dev, R_ZX, s0),
            ]

        def fwd_copies(k):
            off = k * SUB
            return [
                copy(rows(asm.at[s1], p, off, SUB),
                     rows(asm.at[s1], p, off, SUB),
                     3 + k, right_dev, R_FL0 + k, s1),
                copy(rows(asm.at[s1], p, off, SUB),
                     rows(asm.at[s1], p, off, SUB),
                     5 + k, left_dev, R_FR0 + k, s1),
            ]

        def half_copies():
            return [
                copy(rows(asm.at[s2], plft, Z_EXTRA, RING_A),
                     rows(asm.at[s2], plft, Z_EXTRA, RING_A),
                     7, right_dev, R_HL, s2),
                copy(rows(asm.at[s2], pr, Z_EXTRA + RING_A, RING_B),
                     rows(asm.at[s2], pr, Z_EXTRA + RING_A, RING_B),
                     8, left_dev, R_HR, s2),
            ]

        @pl.when(c < N_BLOCKS)
        def _():
            for r in z_copies():
                r.start()

        @pl.when(jnp.logical_and(c >= 1, c <= N_BLOCKS))
        def _():
            copy(rows(asm.at[s1], p, 0, SUB),
                 rows(asm.at[s1], p, 0, SUB),
                 0, zpeer_dev, R_Z0, s1).wait_recv()
            for r in fwd_copies(0):
                r.start()
            copy(rows(asm.at[s1], p, SUB, SUB),
                 rows(asm.at[s1], p, SUB, SUB),
                 0, zpeer_dev, R_Z1, s1).wait_recv()
            for r in fwd_copies(1):
                r.start()

        @pl.when(jnp.logical_and(c >= 2, c <= N_BLOCKS + 1))
        def _():
            copy(rows(asm.at[s2], plft, 0, SUB),
                 rows(asm.at[s2], plft, 0, SUB),
                 0, left_dev, R_FL0, s2).wait_recv()
            copy(rows(asm.at[s2], plft, SUB, SUB),
                 rows(asm.at[s2], plft, SUB, SUB),
                 0, left_dev, R_FL1, s2).wait_recv()
            copy(rows(asm.at[s2], pr, 0, SUB),
                 rows(asm.at[s2], pr, 0, SUB),
                 0, right_dev, R_FR0, s2).wait_recv()
            copy(rows(asm.at[s2], pr, SUB, SUB),
                 rows(asm.at[s2], pr, SUB, SUB),
                 0, right_dev, R_FR1, s2).wait_recv()
            for r in half_copies():
                r.start()

        @pl.when(jnp.logical_and(c >= 3, c <= N_BLOCKS + 2))
        def _():
            copy(rows(asm.at[s3], popp, 0, Z_EXTRA),
                 rows(asm.at[s3], popp, 0, Z_EXTRA),
                 0, zpeer_dev, R_ZX, s3).wait_recv()
            copy(rows(asm.at[s3], popp, Z_EXTRA, RING_A),
                 rows(asm.at[s3], popp, Z_EXTRA, RING_A),
                 0, left_dev, R_HL, s3).wait_recv()
            copy(rows(asm.at[s3], popp, Z_EXTRA + RING_A, RING_B),
                 rows(asm.at[s3], popp, Z_EXTRA + RING_A, RING_B),
                 0, right_dev, R_HR, s3).wait_recv()
            y = partial_cmp_ref[...] + resid_ref[...] + asm[s3]
            inv = lax.rsqrt(jnp.mean(y * y, axis=-1, keepdims=True) + EPS)
            out_ref[...] = y * inv * gamma_ref[...]

        @pl.when(c < N_BLOCKS)
        def _():
            for r in z_copies():
                r.wait_send()

        @pl.when(jnp.logical_and(c >= 1, c <= N_BLOCKS))
        def _():
            for k in (0, 1):
                for r in fwd_copies(k):
                    r.wait_send()

        @pl.when(jnp.logical_and(c >= 2, c <= N_BLOCKS + 1))
        def _():
            for r in half_copies():
                r.wait_send()

    def send_idx(c):
        return (jnp.minimum(c, N_BLOCKS - 1), 0)

    def cmp_idx(c):
        return (jnp.clip(c - 3, 0, N_BLOCKS - 1), 0)

    return pl.pallas_call(
        body,
        grid=(N_BLOCKS + 3,),
        in_specs=[
            pl.BlockSpec((BLOCK_M, D), send_idx),
            pl.BlockSpec((BLOCK_M, D), cmp_idx),
            pl.BlockSpec((1, D), lambda c: (0, 0)),
            pl.BlockSpec((BLOCK_M, D), cmp_idx),
        ],
        out_specs=pl.BlockSpec((BLOCK_M, D), cmp_idx),
        out_shape=jax.ShapeDtypeStruct((M, D), jnp.float32),
        scratch_shapes=[
            pltpu.VMEM((N_SLOTS, BLOCK_M, D), jnp.float32),
            pltpu.SemaphoreType.DMA((N_FLOWS,)),
            pltpu.SemaphoreType.DMA((N_SLOTS, N_FLOWS)),
        ],
        compiler_params=pltpu.CompilerParams(
            vmem_limit_bytes=100 * 1024 * 1024,
        ),
    )(partial, resid, gamma, partial)
